# baseline (speedup 1.0000x reference)
import jax
import jax.numpy as jnp
from jax import lax
from jax.experimental import pallas as pl
from jax.experimental.pallas import tpu as pltpu
from jax.experimental.pallas import tpu_sc as plsc

N = 10000
E = 320000
D = 128

LANES = 128
EROWS = E // LANES
NC = 2
NS = 16
NW = NC * NS
RPT = 80
CHUNK = 40
EPAD = RPT * NW
NPAD = 10240
NPT = NPAD // NS

_MESH = plsc.VectorSubcoreMesh(core_axis_name="c", subcore_axis_name="s")


def _sc_agg_body(h_hbm, src_hbm, dst_hbm,
                 part_hbm, src_v, dst_v, rows0, rows1, acc_sh,
                 sem0, sem1, sems0, sems1):
    cid = lax.axis_index("c")
    sid = lax.axis_index("s")
    wid = cid * NS + sid

    @pl.loop(0, LANES)
    def _(r):
        @pl.loop(0, D // 16)
        def _(c):
            rows0[r, pl.ds(c * 16, 16)] = jnp.zeros((16,), jnp.float32)

    for k in range(NPT // LANES):
        pltpu.make_async_copy(
            rows0, acc_sh.at[pl.ds(sid * NPT + k * LANES, LANES)], sems0).start()
    for k in range(NPT // LANES):
        pltpu.make_async_copy(
            rows0, acc_sh.at[pl.ds(sid * NPT + k * LANES, LANES)], sems0).wait()

    base = wid * RPT
    plsc.subcore_barrier()

    pltpu.sync_copy(src_hbm.at[pl.ds(base, CHUNK)], src_v)
    pltpu.sync_copy(dst_hbm.at[pl.ds(base, CHUNK)], dst_v)
    pltpu.make_async_copy(h_hbm.at[src_v.at[0]], rows0, sem0).start()
    pltpu.make_async_copy(h_hbm.at[src_v.at[1]], rows1, sem1).start()

    @pl.loop(0, RPT, step=2)
    def _(j):
        jc = j % CHUNK
        pltpu.make_async_copy(h_hbm.at[src_v.at[jc]], rows0, sem0).wait()
        pltpu.make_async_copy(rows0, acc_sh.at[dst_v.at[jc]], sems0).start(add=True)
        pltpu.make_async_copy(h_hbm.at[src_v.at[jc + 1]], rows1, sem1).wait()
        pltpu.make_async_copy(rows1, acc_sh.at[dst_v.at[jc + 1]], sems1).start(add=True)

        @pl.when(jc + 2 >= CHUNK)
        def _():
            @pl.when(j + 2 < RPT)
            def _():
                nb = pl.multiple_of(base + j + 2, 8)
                pltpu.sync_copy(src_hbm.at[pl.ds(nb, CHUNK)], src_v)
                pltpu.sync_copy(dst_hbm.at[pl.ds(nb, CHUNK)], dst_v)

        jn = (j + 2) % CHUNK
        pltpu.make_async_copy(rows0, acc_sh.at[dst_v.at[jc]], sems0).wait()

        @pl.when(j + 2 < RPT)
        def _():
            pltpu.make_async_copy(h_hbm.at[src_v.at[jn]], rows0, sem0).start()

        pltpu.make_async_copy(rows1, acc_sh.at[dst_v.at[jc + 1]], sems1).wait()

        @pl.when(j + 3 < RPT)
        def _():
            pltpu.make_async_copy(h_hbm.at[src_v.at[jn + 1]], rows1, sem1).start()

    plsc.subcore_barrier()

    pltpu.sync_copy(acc_sh.at[pl.ds(sid * NPT, NPT)],
                    part_hbm.at[cid, pl.ds(sid * NPT, NPT)])


_sc_agg = pl.kernel(
    _sc_agg_body,
    out_type=[jax.ShapeDtypeStruct((NC, NPAD, D), jnp.float32)],
    mesh=_MESH,
    scratch_types=[
        pltpu.VMEM((CHUNK, LANES), jnp.int32),
        pltpu.VMEM((CHUNK, LANES), jnp.int32),
        pltpu.VMEM((LANES, D), jnp.float32),
        pltpu.VMEM((LANES, D), jnp.float32),
        pltpu.VMEM_SHARED((NPAD, D), jnp.float32),
        pltpu.SemaphoreType.DMA,
        pltpu.SemaphoreType.DMA,
        pltpu.SemaphoreType.DMA,
        pltpu.SemaphoreType.DMA,
    ])


def _sc_deg_body(dst_hbm, zero_hbm, ones_hbm,
                 degp_hbm, dst_v, ones_v, deg_sh, sem0):
    cid = lax.axis_index("c")
    sid = lax.axis_index("s")
    wid = cid * NS + sid

    pltpu.sync_copy(zero_hbm.at[pl.ds(sid * NPT, NPT)],
                    deg_sh.at[pl.ds(sid * NPT, NPT)])
    pltpu.sync_copy(ones_hbm, ones_v)
    pltpu.sync_copy(dst_hbm.at[pl.ds(wid * RPT, RPT)], dst_v)

    plsc.subcore_barrier()

    @pl.loop(0, RPT, step=8)
    def _(j):
        for k in range(8):
            pltpu.make_async_copy(
                ones_v, deg_sh.at[dst_v.at[j + k]], sem0).start(add=True)
        for k in range(8):
            pltpu.make_async_copy(
                ones_v, deg_sh.at[dst_v.at[j + k]], sem0).wait()

    plsc.subcore_barrier()

    pltpu.sync_copy(deg_sh.at[pl.ds(sid * NPT, NPT)],
                    degp_hbm.at[cid, pl.ds(sid * NPT, NPT)])


_sc_deg = pl.kernel(
    _sc_deg_body,
    out_type=[jax.ShapeDtypeStruct((NC, NPAD, D), jnp.float32)],
    mesh=_MESH,
    scratch_types=[
        pltpu.VMEM((RPT, LANES), jnp.int32),
        pltpu.VMEM((LANES, D), jnp.float32),
        pltpu.VMEM_SHARED((NPAD, D), jnp.float32),
        pltpu.SemaphoreType.DMA,
    ])


def _dotT(x, w):
    return lax.dot_general(x, w, (((1,), (1,)), ((), ())),
                           preferred_element_type=jnp.float32)


def _in_proj_body(x_ref, w_ref, b_ref, o_ref):
    o_ref[...] = _dotT(x_ref[...], w_ref[...]) + b_ref[...]


_in_proj = pl.pallas_call(
    _in_proj_body,
    out_shape=jax.ShapeDtypeStruct((N, D), jnp.float32),
)


def _combine_body(pp, dgp, o):
    deg = dgp[0, :N, :1] + dgp[1, :N, :1]
    o[...] = (pp[0, :N] + pp[1, :N]) / jnp.maximum(deg, 1.0)


_combine = pl.pallas_call(
    _combine_body, out_shape=jax.ShapeDtypeStruct((N, D), jnp.float32))


def _layer_factory(final: bool):
    def body(*refs):
        if final:
            (mean_msg, h, wl, bl, wr, al, gm, be, wo, bo, o) = refs
        else:
            (mean_msg, h, wl, bl, wr, al, gm, be, o) = refs
        y = _dotT(mean_msg[...], wl[...]) + bl[...] + _dotT(h[...], wr[...])
        m = jnp.mean(y, axis=0, keepdims=True)
        sub = y - al[...] * m
        var = jnp.mean(sub * sub, axis=0, keepdims=True)
        z = gm[...] * sub * lax.rsqrt(var + 1e-5) + be[...]
        hn = jnp.where(z > 0, z, 0.1 * z)
        if final:
            o[...] = _dotT(hn, wo[...]) + bo[...]
        else:
            o[...] = hn

    return pl.pallas_call(
        body, out_shape=jax.ShapeDtypeStruct((N, D), jnp.float32))


_layer = _layer_factory(final=False)
_layer_final = _layer_factory(final=True)


def kernel(x, edge_index, W_in, b_in,
           Wl1, bl1, Wr1, a1, g1, be1,
           Wl2, bl2, Wr2, a2, g2, be2,
           Wl3, bl3, Wr3, a3, g3, be3,
           W_out, b_out):
    npad_edges = EPAD * LANES - E
    ei = edge_index.astype(jnp.int32)
    src2 = jnp.concatenate(
        [ei[0], jnp.zeros((npad_edges,), jnp.int32)]).reshape(EPAD, LANES)
    dst2 = jnp.concatenate(
        [ei[1], jnp.full((npad_edges,), N, jnp.int32)]).reshape(EPAD, LANES)
    zeros_nd = jnp.zeros((NPAD, D), jnp.float32)
    ones_rows = jnp.ones((LANES, D), jnp.float32)

    r1 = lambda v: v.reshape(1, D)

    h = _in_proj(x, W_in, r1(b_in))
    (degp,) = _sc_deg(dst2, zeros_nd, ones_rows)
    (part,) = _sc_agg(h, src2, dst2)
    mm = _combine(part, degp)
    h = _layer(mm, h, Wl1, r1(bl1), Wr1, r1(a1), r1(g1), r1(be1))
    (part,) = _sc_agg(h, src2, dst2)
    mm = _combine(part, degp)
    h = _layer(mm, h, Wl2, r1(bl2), Wr2, r1(a2), r1(g2), r1(be2))
    (part,) = _sc_agg(h, src2, dst2)
    mm = _combine(part, degp)
    out = _layer_final(mm, h, Wl3, r1(bl3), Wr3, r1(a3), r1(g3),
                       r1(be3), W_out, r1(b_out))
    return out


# speedup vs baseline: 1.0262x; 1.0262x over previous
import jax
import jax.numpy as jnp
from jax import lax
from jax.experimental import pallas as pl
from jax.experimental.pallas import tpu as pltpu
from jax.experimental.pallas import tpu_sc as plsc

N = 10000
E = 320000
D = 128

LANES = 128
EROWS = E // LANES
NC = 2
NS = 16
NW = NC * NS
RPT = 80
CHUNK = 40
EPAD = RPT * NW
NPAD = 10240
NPT = NPAD // NS

_MESH = plsc.VectorSubcoreMesh(core_axis_name="c", subcore_axis_name="s")


def _sc_agg_body(h_hbm, src_hbm, dst_hbm,
                 part_hbm, src_v, dst_v, rows0, rows1, acc_sh,
                 sem0, sem1, sems0, sems1):
    cid = lax.axis_index("c")
    sid = lax.axis_index("s")
    wid = cid * NS + sid

    @pl.loop(0, LANES)
    def _(r):
        @pl.loop(0, D // 16)
        def _(c):
            rows0[r, pl.ds(c * 16, 16)] = jnp.zeros((16,), jnp.float32)

    for k in range(NPT // LANES):
        pltpu.make_async_copy(
            rows0, acc_sh.at[pl.ds(sid * NPT + k * LANES, LANES)], sems0).start()
    for k in range(NPT // LANES):
        pltpu.make_async_copy(
            rows0, acc_sh.at[pl.ds(sid * NPT + k * LANES, LANES)], sems0).wait()

    base = wid * RPT
    plsc.subcore_barrier()

    @pl.loop(0, RPT // CHUNK)
    def _(c):
        cb = base + c * CHUNK
        pltpu.sync_copy(src_hbm.at[pl.ds(cb, CHUNK)], src_v)
        pltpu.sync_copy(dst_hbm.at[pl.ds(cb, CHUNK)], dst_v)
        pltpu.make_async_copy(h_hbm.at[src_v.at[0]], rows0, sem0).start()

        @pl.loop(0, CHUNK, step=2)
        def _(j):
            pltpu.make_async_copy(h_hbm.at[src_v.at[j + 1]], rows1, sem1).start()
            pltpu.make_async_copy(h_hbm.at[src_v.at[j]], rows0, sem0).wait()
            pltpu.sync_copy(rows0, acc_sh.at[dst_v.at[j]], add=True)

            @pl.when(j + 2 < CHUNK)
            def _():
                pltpu.make_async_copy(h_hbm.at[src_v.at[j + 2]], rows0, sem0).start()

            pltpu.make_async_copy(h_hbm.at[src_v.at[j + 1]], rows1, sem1).wait()
            pltpu.sync_copy(rows1, acc_sh.at[dst_v.at[j + 1]], add=True)

    plsc.subcore_barrier()

    pltpu.sync_copy(acc_sh.at[pl.ds(sid * NPT, NPT)],
                    part_hbm.at[cid, pl.ds(sid * NPT, NPT)])


_sc_agg = pl.kernel(
    _sc_agg_body,
    out_type=[jax.ShapeDtypeStruct((NC, NPAD, D), jnp.float32)],
    mesh=_MESH,
    scratch_types=[
        pltpu.VMEM((CHUNK, LANES), jnp.int32),
        pltpu.VMEM((CHUNK, LANES), jnp.int32),
        pltpu.VMEM((LANES, D), jnp.float32),
        pltpu.VMEM((LANES, D), jnp.float32),
        pltpu.VMEM_SHARED((NPAD, D), jnp.float32),
        pltpu.SemaphoreType.DMA,
        pltpu.SemaphoreType.DMA,
        pltpu.SemaphoreType.DMA,
        pltpu.SemaphoreType.DMA,
    ])


def _sc_deg_body(dst_hbm, zero_hbm, ones_hbm,
                 degp_hbm, dst_v, ones_v, deg_sh, sem0):
    cid = lax.axis_index("c")
    sid = lax.axis_index("s")
    wid = cid * NS + sid

    pltpu.sync_copy(zero_hbm.at[pl.ds(sid * NPT, NPT)],
                    deg_sh.at[pl.ds(sid * NPT, NPT)])
    pltpu.sync_copy(ones_hbm, ones_v)
    pltpu.sync_copy(dst_hbm.at[pl.ds(wid * RPT, RPT)], dst_v)

    plsc.subcore_barrier()

    @pl.loop(0, RPT, step=8)
    def _(j):
        for k in range(8):
            pltpu.make_async_copy(
                ones_v, deg_sh.at[dst_v.at[j + k]], sem0).start(add=True)
        for k in range(8):
            pltpu.make_async_copy(
                ones_v, deg_sh.at[dst_v.at[j + k]], sem0).wait()

    plsc.subcore_barrier()

    pltpu.sync_copy(deg_sh.at[pl.ds(sid * NPT, NPT)],
                    degp_hbm.at[cid, pl.ds(sid * NPT, NPT)])


_sc_deg = pl.kernel(
    _sc_deg_body,
    out_type=[jax.ShapeDtypeStruct((NC, NPAD, D), jnp.float32)],
    mesh=_MESH,
    scratch_types=[
        pltpu.VMEM((RPT, LANES), jnp.int32),
        pltpu.VMEM((LANES, D), jnp.float32),
        pltpu.VMEM_SHARED((NPAD, D), jnp.float32),
        pltpu.SemaphoreType.DMA,
    ])


def _dotT(x, w):
    return lax.dot_general(x, w, (((1,), (1,)), ((), ())),
                           preferred_element_type=jnp.float32)


def _in_proj_body(x_ref, w_ref, b_ref, o_ref):
    o_ref[...] = _dotT(x_ref[...], w_ref[...]) + b_ref[...]


_in_proj = pl.pallas_call(
    _in_proj_body,
    out_shape=jax.ShapeDtypeStruct((N, D), jnp.float32),
)


def _combine_body(pp, dgp, o):
    deg = dgp[0, :N, :1] + dgp[1, :N, :1]
    o[...] = (pp[0, :N] + pp[1, :N]) / jnp.maximum(deg, 1.0)


_combine = pl.pallas_call(
    _combine_body, out_shape=jax.ShapeDtypeStruct((N, D), jnp.float32))


def _layer_factory(final: bool):
    def body(*refs):
        if final:
            (mean_msg, h, wl, bl, wr, al, gm, be, wo, bo, o) = refs
        else:
            (mean_msg, h, wl, bl, wr, al, gm, be, o) = refs
        y = _dotT(mean_msg[...], wl[...]) + bl[...] + _dotT(h[...], wr[...])
        m = jnp.mean(y, axis=0, keepdims=True)
        sub = y - al[...] * m
        var = jnp.mean(sub * sub, axis=0, keepdims=True)
        z = gm[...] * sub * lax.rsqrt(var + 1e-5) + be[...]
        hn = jnp.where(z > 0, z, 0.1 * z)
        if final:
            o[...] = _dotT(hn, wo[...]) + bo[...]
        else:
            o[...] = hn

    return pl.pallas_call(
        body, out_shape=jax.ShapeDtypeStruct((N, D), jnp.float32))


_layer = _layer_factory(final=False)
_layer_final = _layer_factory(final=True)


def kernel(x, edge_index, W_in, b_in,
           Wl1, bl1, Wr1, a1, g1, be1,
           Wl2, bl2, Wr2, a2, g2, be2,
           Wl3, bl3, Wr3, a3, g3, be3,
           W_out, b_out):
    npad_edges = EPAD * LANES - E
    ei = edge_index.astype(jnp.int32)
    src2 = jnp.concatenate(
        [ei[0], jnp.zeros((npad_edges,), jnp.int32)]).reshape(EPAD, LANES)
    dst2 = jnp.concatenate(
        [ei[1], jnp.full((npad_edges,), N, jnp.int32)]).reshape(EPAD, LANES)
    zeros_nd = jnp.zeros((NPAD, D), jnp.float32)
    ones_rows = jnp.ones((LANES, D), jnp.float32)

    r1 = lambda v: v.reshape(1, D)

    h = _in_proj(x, W_in, r1(b_in))
    (degp,) = _sc_deg(dst2, zeros_nd, ones_rows)
    (part,) = _sc_agg(h, src2, dst2)
    mm = _combine(part, degp)
    h = _layer(mm, h, Wl1, r1(bl1), Wr1, r1(a1), r1(g1), r1(be1))
    (part,) = _sc_agg(h, src2, dst2)
    mm = _combine(part, degp)
    h = _layer(mm, h, Wl2, r1(bl2), Wr2, r1(a2), r1(g2), r1(be2))
    (part,) = _sc_agg(h, src2, dst2)
    mm = _combine(part, degp)
    out = _layer_final(mm, h, Wl3, r1(bl3), Wr3, r1(a3), r1(g3),
                       r1(be3), W_out, r1(b_out))
    return out
